# Initial kernel scaffold; baseline (speedup 1.0000x reference)
import functools

import jax
import jax.numpy as jnp
from jax import lax
from jax.experimental import pallas as pl
from jax.experimental.pallas import tpu as pltpu
from jax.experimental.pallas import tpu_sc as plsc

N = 16384
B = 64
NC = 2
NS = 16
NW = NC * NS
CHUNK = 128


def _xpose_body(x_ref, o_ref):
    o_ref[...] = x_ref[...].T


def _transpose_in(x):
    blk = 2048
    return pl.pallas_call(
        _xpose_body,
        grid=(N // blk,),
        in_specs=[pl.BlockSpec((B, blk), lambda i: (0, i))],
        out_specs=pl.BlockSpec((blk, B), lambda i: (i, 0)),
        out_shape=jax.ShapeDtypeStruct((N, B), jnp.float32),
    )(x)


def _merge_body(p_ref, o_ref):
    o_ref[...] = (p_ref[0] + p_ref[1]).T


def _merge_out(p):
    blk = 2048
    return pl.pallas_call(
        _merge_body,
        grid=(N // blk,),
        in_specs=[pl.BlockSpec((NC, blk, B), lambda i: (0, i, 0))],
        out_specs=pl.BlockSpec((B, blk), lambda i: (0, i)),
        out_shape=jax.ShapeDtypeStruct((B, N), jnp.float32),
    )(p)


def _splat(v16, i):
    return jnp.broadcast_to(v16[i], (16,))


def _sc_spmm(xt, cols2d, rows2d, vals2d, k_chunks):
    mesh = plsc.VectorSubcoreMesh(core_axis_name="c", subcore_axis_name="s")
    rows_per_tile = N // NS

    @functools.partial(
        pl.kernel,
        out_type=jax.ShapeDtypeStruct((NC, N, B), jnp.float32),
        mesh=mesh,
        scratch_types=[
            pltpu.VMEM((k_chunks, CHUNK), jnp.int32),
            pltpu.VMEM((k_chunks, CHUNK), jnp.int32),
            pltpu.VMEM((k_chunks, CHUNK), jnp.float32),
            pltpu.VMEM((CHUNK, B), jnp.float32),
            pltpu.VMEM_SHARED((N, B), jnp.float32),
            pltpu.SemaphoreType.DMA,
        ],
    )
    def sc_kernel(xt_hbm, cols_hbm, rows_hbm, vals_hbm, out_hbm,
                  cols_v, rows_v, vals_v, gath_v, acc, sem):
        c = lax.axis_index("c")
        s = lax.axis_index("s")
        wid = c * NS + s
        base = wid * k_chunks

        pltpu.sync_copy(cols_hbm.at[pl.ds(base, k_chunks)], cols_v)
        pltpu.sync_copy(rows_hbm.at[pl.ds(base, k_chunks)], rows_v)
        pltpu.sync_copy(vals_hbm.at[pl.ds(base, k_chunks)], vals_v)

        @pl.loop(0, CHUNK)
        def _(i):
            for kk in range(B // 16):
                gath_v[i, pl.ds(kk * 16, 16)] = jnp.zeros((16,), jnp.float32)

        zbase = s * rows_per_tile

        @pl.loop(0, rows_per_tile // CHUNK)
        def _(q):
            pltpu.sync_copy(gath_v, acc.at[pl.ds(zbase + q * CHUNK, CHUNK)])

        plsc.subcore_barrier()

        @pl.loop(0, k_chunks)
        def _(j):
            pltpu.async_copy(xt_hbm.at[cols_v.at[j]], gath_v, sem).wait()

            @pl.loop(0, CHUNK // 16)
            def _(g):
                v16 = vals_v[j, pl.ds(g * 16, 16)]
                for i in range(16):
                    sp = _splat(v16, i)
                    r = g * 16 + i
                    for kk in range(B // 16):
                        sl = pl.ds(kk * 16, 16)
                        gath_v[r, sl] = gath_v[r, sl] * sp

            pltpu.sync_copy(gath_v, acc.at[rows_v.at[j]], add=True)

        plsc.subcore_barrier()

        pltpu.sync_copy(acc.at[pl.ds(zbase, rows_per_tile)],
                        out_hbm.at[c].at[pl.ds(zbase, rows_per_tile)])

    return sc_kernel(xt, cols2d, rows2d, vals2d)


@jax.jit
def kernel(x_affine, W_rows, W_cols, W_vals):
    nnz = W_rows.shape[0]
    per_step = NW * CHUNK
    k_chunks = (nnz + per_step - 1) // per_step
    nnz_pad = k_chunks * per_step
    pad = nnz_pad - nnz

    pad_idx = (jnp.arange(pad, dtype=jnp.int32) * 101) % N
    cols2d = jnp.concatenate([W_cols.astype(jnp.int32), pad_idx]).reshape(-1, CHUNK)
    rows2d = jnp.concatenate([W_rows.astype(jnp.int32), pad_idx]).reshape(-1, CHUNK)
    vals2d = jnp.concatenate(
        [W_vals, jnp.zeros((pad,), jnp.float32)]).reshape(-1, CHUNK)

    xt = _transpose_in(x_affine)
    partials = _sc_spmm(xt, cols2d, rows2d, vals2d, k_chunks)
    return _merge_out(partials)


# speedup vs baseline: 7.3139x; 7.3139x over previous
import functools

import jax
import jax.numpy as jnp
from jax import lax
from jax.experimental import pallas as pl
from jax.experimental.pallas import tpu as pltpu
from jax.experimental.pallas import tpu_sc as plsc

N = 16384
B = 64
NC = 2
NS = 16
NW = NC * NS
CHUNK = 128


def _xpose_body(x_ref, o_ref):
    o_ref[...] = x_ref[...].T


def _transpose_in(x):
    blk = 2048
    return pl.pallas_call(
        _xpose_body,
        grid=(N // blk,),
        in_specs=[pl.BlockSpec((B, blk), lambda i: (0, i))],
        out_specs=pl.BlockSpec((blk, B), lambda i: (i, 0)),
        out_shape=jax.ShapeDtypeStruct((N, B), jnp.float32),
    )(x)


def _merge_body(p_ref, o_ref):
    o_ref[...] = (p_ref[0] + p_ref[1]).T


def _merge_out(p):
    blk = 2048
    return pl.pallas_call(
        _merge_body,
        grid=(N // blk,),
        in_specs=[pl.BlockSpec((NC, blk, B), lambda i: (0, i, 0))],
        out_specs=pl.BlockSpec((B, blk), lambda i: (0, i)),
        out_shape=jax.ShapeDtypeStruct((B, N), jnp.float32),
    )(p)


def _splat(v16, i):
    return jnp.broadcast_to(v16[i], (16,))


def _sc_spmm(xt, cols2d, rows2d, vals2d, k_chunks):
    mesh = plsc.VectorSubcoreMesh(core_axis_name="c", subcore_axis_name="s")
    rows_per_tile = N // NS

    @functools.partial(
        pl.kernel,
        out_type=jax.ShapeDtypeStruct((NC, N, B), jnp.float32),
        mesh=mesh,
        scratch_types=[
            pltpu.VMEM((k_chunks, CHUNK), jnp.int32),
            pltpu.VMEM((k_chunks, CHUNK), jnp.int32),
            pltpu.VMEM((k_chunks, CHUNK), jnp.float32),
            pltpu.VMEM((CHUNK, B), jnp.float32),
            pltpu.VMEM_SHARED((N, B), jnp.float32),
            pltpu.SemaphoreType.DMA,
        ],
        compiler_params=pltpu.CompilerParams(use_tc_tiling_on_sc=False),
    )
    def sc_kernel(xt_hbm, cols_hbm, rows_hbm, vals_hbm, out_hbm,
                  cols_v, rows_v, vals_v, gath_v, acc, sem):
        c = lax.axis_index("c")
        s = lax.axis_index("s")
        wid = c * NS + s

        pltpu.sync_copy(cols_hbm.at[wid], cols_v)
        pltpu.sync_copy(rows_hbm.at[wid], rows_v)
        pltpu.sync_copy(vals_hbm.at[wid], vals_v)

        @pl.loop(0, CHUNK)
        def _(i):
            for kk in range(B // 16):
                gath_v[i, pl.ds(kk * 16, 16)] = jnp.zeros((16,), jnp.float32)

        zbase = s * rows_per_tile

        @pl.loop(0, rows_per_tile // CHUNK)
        def _(q):
            pltpu.sync_copy(gath_v, acc.at[pl.ds(zbase + q * CHUNK, CHUNK)])

        plsc.subcore_barrier()

        @pl.loop(0, k_chunks)
        def _(j):
            pltpu.async_copy(xt_hbm.at[cols_v.at[j]], gath_v, sem).wait()

            @pl.loop(0, CHUNK // 16)
            def _(g):
                v16 = vals_v[j, pl.ds(g * 16, 16)]
                for i in range(16):
                    sp = _splat(v16, i)
                    r = g * 16 + i
                    for kk in range(B // 16):
                        sl = pl.ds(kk * 16, 16)
                        gath_v[r, sl] = gath_v[r, sl] * sp

            pltpu.sync_copy(gath_v, acc.at[rows_v.at[j]], add=True)

        plsc.subcore_barrier()

        pltpu.sync_copy(acc.at[pl.ds(zbase, rows_per_tile)],
                        out_hbm.at[c].at[pl.ds(zbase, rows_per_tile)])

    return sc_kernel(xt, cols2d, rows2d, vals2d)


@jax.jit
def kernel(x_affine, W_rows, W_cols, W_vals):
    nnz = W_rows.shape[0]
    per_step = NW * CHUNK
    k_chunks = (nnz + per_step - 1) // per_step
    nnz_pad = k_chunks * per_step
    pad = nnz_pad - nnz

    pad_idx = (jnp.arange(pad, dtype=jnp.int32) * 101) % N
    shape3 = (NW, k_chunks, CHUNK)
    cols2d = jnp.concatenate([W_cols.astype(jnp.int32), pad_idx]).reshape(shape3)
    rows2d = jnp.concatenate([W_rows.astype(jnp.int32), pad_idx]).reshape(shape3)
    vals2d = jnp.concatenate(
        [W_vals, jnp.zeros((pad,), jnp.float32)]).reshape(shape3)

    xt = _transpose_in(x_affine)
    partials = _sc_spmm(xt, cols2d, rows2d, vals2d, k_chunks)
    return _merge_out(partials)


# speedup vs baseline: 18.4422x; 2.5215x over previous
import functools

import jax
import jax.numpy as jnp
from jax import lax
from jax.experimental import pallas as pl
from jax.experimental.pallas import tpu as pltpu
from jax.experimental.pallas import tpu_sc as plsc

N = 16384
B = 64
NC = 2
NS = 16
NW = NC * NS
CHUNK = 128


def _xpose_body(x_ref, o_ref):
    o_ref[...] = x_ref[...].T


def _transpose_in(x):
    blk = 2048
    return pl.pallas_call(
        _xpose_body,
        grid=(N // blk,),
        in_specs=[pl.BlockSpec((B, blk), lambda i: (0, i))],
        out_specs=pl.BlockSpec((blk, B), lambda i: (i, 0)),
        out_shape=jax.ShapeDtypeStruct((N, B), jnp.float32),
    )(x)


def _merge_body(p_ref, o_ref):
    o_ref[...] = (p_ref[0] + p_ref[1]).T


def _merge_out(p):
    blk = 2048
    return pl.pallas_call(
        _merge_body,
        grid=(N // blk,),
        in_specs=[pl.BlockSpec((NC, blk, B), lambda i: (0, i, 0))],
        out_specs=pl.BlockSpec((B, blk), lambda i: (0, i)),
        out_shape=jax.ShapeDtypeStruct((B, N), jnp.float32),
    )(p)


def _splat(v16, i):
    return jnp.broadcast_to(v16[i], (16,))


def _sc_spmm(xt, cols2d, rows2d, vals2d, k_chunks):
    mesh = plsc.VectorSubcoreMesh(core_axis_name="c", subcore_axis_name="s")
    rows_per_tile = N // NS

    @functools.partial(
        pl.kernel,
        out_type=jax.ShapeDtypeStruct((NC, N, B), jnp.float32),
        mesh=mesh,
        scratch_types=[
            pltpu.VMEM((k_chunks, CHUNK), jnp.int32),
            pltpu.VMEM((k_chunks, CHUNK), jnp.int32),
            pltpu.VMEM((k_chunks, CHUNK), jnp.float32),
            pltpu.VMEM((2, CHUNK, B), jnp.float32),
            pltpu.VMEM((2, CHUNK, B), jnp.float32),
            pltpu.VMEM_SHARED((N, B), jnp.float32),
            pltpu.SemaphoreType.DMA,
            pltpu.SemaphoreType.DMA,
            pltpu.SemaphoreType.DMA,
            pltpu.SemaphoreType.DMA,
            pltpu.SemaphoreType.DMA,
            pltpu.SemaphoreType.DMA,
        ],
        compiler_params=pltpu.CompilerParams(use_tc_tiling_on_sc=False),
    )
    def sc_kernel(xt_hbm, cols_hbm, rows_hbm, vals_hbm, out_hbm,
                  cols_v, rows_v, vals_v, gath_v, scat_v, acc,
                  sem_st, zsem, gsem0, gsem1, ssem0, ssem1):
        gsem = (gsem0, gsem1)
        ssem = (ssem0, ssem1)
        c = lax.axis_index("c")
        s = lax.axis_index("s")
        wid = c * NS + s

        pltpu.async_copy(cols_hbm.at[wid], cols_v, sem_st)
        pltpu.async_copy(rows_hbm.at[wid], rows_v, sem_st)
        pltpu.async_copy(vals_hbm.at[wid], vals_v, sem_st)

        zb = scat_v.at[0]

        @pl.loop(0, CHUNK)
        def _(i):
            for kk in range(B // 16):
                zb[i, pl.ds(kk * 16, 16)] = jnp.zeros((16,), jnp.float32)

        zbase = s * rows_per_tile
        n_z = rows_per_tile // CHUNK
        for q in range(n_z):
            pltpu.async_copy(zb, acc.at[pl.ds(zbase + q * CHUNK, CHUNK)], zsem)

        pltpu.make_async_copy(cols_hbm.at[wid], cols_v, sem_st).wait()
        pltpu.make_async_copy(rows_hbm.at[wid], rows_v, sem_st).wait()
        pltpu.make_async_copy(vals_hbm.at[wid], vals_v, sem_st).wait()

        def start_gather(j, b):
            pltpu.async_copy(xt_hbm.at[cols_v.at[j]], gath_v.at[b], gsem[b])

        def wait_gather(b):
            pltpu.make_async_copy(xt_hbm.at[cols_v.at[0]], gath_v.at[b],
                                  gsem[b]).wait()

        def start_scatter(j, b):
            pltpu.async_copy(scat_v.at[b], acc.at[rows_v.at[j]], ssem[b],
                             add=True)

        def wait_scatter(b):
            pltpu.make_async_copy(xt_hbm.at[cols_v.at[0]], scat_v.at[b],
                                  ssem[b]).wait()

        start_gather(0, 0)
        start_gather(1, 1)

        for q in range(n_z):
            pltpu.make_async_copy(zb, acc.at[pl.ds(zbase, CHUNK)], zsem).wait()
        plsc.subcore_barrier()

        @pl.loop(0, k_chunks, step=2)
        def _(j):
            for b in range(2):
                jj = j + b
                wait_gather(b)

                @pl.when(jj >= 2)
                def _():
                    wait_scatter(b)

                g_b = gath_v.at[b]
                s_b = scat_v.at[b]

                @pl.loop(0, CHUNK // 16)
                def _(g):
                    v16 = vals_v[jj, pl.ds(g * 16, 16)]
                    for i in range(16):
                        sp = _splat(v16, i)
                        r = g * 16 + i
                        for kk in range(B // 16):
                            sl = pl.ds(kk * 16, 16)
                            s_b[r, sl] = g_b[r, sl] * sp

                @pl.when(jj + 2 < k_chunks)
                def _():
                    start_gather(jj + 2, b)

                start_scatter(jj, b)

        wait_scatter(0)
        wait_scatter(1)
        plsc.subcore_barrier()

        pltpu.sync_copy(acc.at[pl.ds(zbase, rows_per_tile)],
                        out_hbm.at[c].at[pl.ds(zbase, rows_per_tile)])

    return sc_kernel(xt, cols2d, rows2d, vals2d)


@jax.jit
def kernel(x_affine, W_rows, W_cols, W_vals):
    nnz = W_rows.shape[0]
    per_step = NW * CHUNK
    k_chunks = (nnz + per_step - 1) // per_step
    nnz_pad = k_chunks * per_step
    pad = nnz_pad - nnz

    pad_idx = (jnp.arange(pad, dtype=jnp.int32) * 101) % N
    shape3 = (NW, k_chunks, CHUNK)
    cols2d = jnp.concatenate([W_cols.astype(jnp.int32), pad_idx]).reshape(shape3)
    rows2d = jnp.concatenate([W_rows.astype(jnp.int32), pad_idx]).reshape(shape3)
    vals2d = jnp.concatenate(
        [W_vals, jnp.zeros((pad,), jnp.float32)]).reshape(shape3)

    xt = _transpose_in(x_affine)
    partials = _sc_spmm(xt, cols2d, rows2d, vals2d, k_chunks)
    return _merge_out(partials)


# speedup vs baseline: 18.7065x; 1.0143x over previous
import functools

import jax
import jax.numpy as jnp
from jax import lax
from jax.experimental import pallas as pl
from jax.experimental.pallas import tpu as pltpu
from jax.experimental.pallas import tpu_sc as plsc

N = 16384
B = 64
NC = 2
NS = 16
NW = NC * NS
CHUNK = 128


def _merge_body(p_ref, o_ref):
    o_ref[...] = (p_ref[0] + p_ref[1]).T


def _merge_out(p):
    blk = 2048
    return pl.pallas_call(
        _merge_body,
        grid=(N // blk,),
        in_specs=[pl.BlockSpec((NC, blk, B), lambda i: (0, i, 0))],
        out_specs=pl.BlockSpec((B, blk), lambda i: (0, i)),
        out_shape=jax.ShapeDtypeStruct((B, N), jnp.float32),
    )(p)


def _splat(v16, i):
    return jnp.broadcast_to(v16[i], (16,))


def _sc_spmm(xt, cols1, rows1, vals1, k_chunks):
    mesh = plsc.VectorSubcoreMesh(core_axis_name="c", subcore_axis_name="s")
    rows_per_tile = N // NS
    t_nnz = k_chunks * CHUNK

    @functools.partial(
        pl.kernel,
        out_type=jax.ShapeDtypeStruct((NC, N, B), jnp.float32),
        mesh=mesh,
        scratch_types=[
            pltpu.VMEM((t_nnz,), jnp.int32),
            pltpu.VMEM((k_chunks, CHUNK), jnp.int32),
            pltpu.VMEM((t_nnz,), jnp.float32),
            pltpu.VMEM((2, CHUNK, B), jnp.float32),
            pltpu.VMEM((2, CHUNK, B), jnp.float32),
            pltpu.VMEM_SHARED((N, B), jnp.float32),
            pltpu.SemaphoreType.DMA,
            pltpu.SemaphoreType.DMA,
            pltpu.SemaphoreType.DMA,
            pltpu.SemaphoreType.DMA,
            pltpu.SemaphoreType.DMA,
            pltpu.SemaphoreType.DMA,
        ],
        compiler_params=pltpu.CompilerParams(use_tc_tiling_on_sc=False),
    )
    def sc_kernel(xt_hbm, cols_hbm, rows_hbm, vals_hbm, out_hbm,
                  cols_v, rows_v, vals_v, gath_v, scat_v, acc,
                  sem_st, zsem, gsem0, gsem1, ssem0, ssem1):
        gsem = (gsem0, gsem1)
        ssem = (ssem0, ssem1)
        c = lax.axis_index("c")
        s = lax.axis_index("s")
        wid = c * NS + s
        base = wid * t_nnz

        pltpu.async_copy(cols_hbm.at[pl.ds(base, t_nnz)], cols_v, sem_st)
        pltpu.async_copy(vals_hbm.at[pl.ds(base, t_nnz)], vals_v, sem_st)

        @pl.loop(0, k_chunks)
        def _(j):
            pltpu.async_copy(rows_hbm.at[pl.ds(base + j * CHUNK, CHUNK)],
                             rows_v.at[j], sem_st)

        zb = scat_v.at[0]

        @pl.loop(0, CHUNK)
        def _(i):
            for kk in range(B // 16):
                zb[i, pl.ds(kk * 16, 16)] = jnp.zeros((16,), jnp.float32)

        zbase = s * rows_per_tile
        n_z = rows_per_tile // CHUNK
        for q in range(n_z):
            pltpu.async_copy(zb, acc.at[pl.ds(zbase + q * CHUNK, CHUNK)], zsem)

        pltpu.make_async_copy(cols_hbm.at[pl.ds(base, t_nnz)], cols_v,
                              sem_st).wait()
        pltpu.make_async_copy(cols_hbm.at[pl.ds(base, t_nnz)], vals_v,
                              sem_st).wait()

        @pl.loop(0, k_chunks)
        def _(j):
            pltpu.make_async_copy(rows_hbm.at[pl.ds(base, CHUNK)],
                                  rows_v.at[0], sem_st).wait()

        def start_gather(j, b):
            pltpu.async_copy(xt_hbm.at[cols_v.at[pl.ds(j * CHUNK, CHUNK)]],
                             gath_v.at[b], gsem[b])

        def wait_gather(b):
            pltpu.make_async_copy(xt_hbm.at[cols_v.at[pl.ds(0, CHUNK)]],
                                  gath_v.at[b], gsem[b]).wait()

        def start_scatter(j, b):
            pltpu.async_copy(scat_v.at[b], acc.at[rows_v.at[j]], ssem[b],
                             add=True)

        def wait_scatter(b):
            pltpu.make_async_copy(xt_hbm.at[cols_v.at[pl.ds(0, CHUNK)]],
                                  scat_v.at[b], ssem[b]).wait()

        start_gather(0, 0)
        start_gather(1, 1)

        for q in range(n_z):
            pltpu.make_async_copy(zb, acc.at[pl.ds(zbase, CHUNK)], zsem).wait()
        plsc.subcore_barrier()

        @pl.loop(0, k_chunks, step=2)
        def _(j):
            for b in range(2):
                jj = j + b
                wait_gather(b)

                @pl.when(jj >= 2)
                def _():
                    wait_scatter(b)

                g_b = gath_v.at[b]
                s_b = scat_v.at[b]

                @pl.loop(0, CHUNK // 16)
                def _(g):
                    v16 = vals_v[pl.ds(jj * CHUNK + g * 16, 16)]
                    for i in range(16):
                        sp = _splat(v16, i)
                        r = g * 16 + i
                        for kk in range(B // 16):
                            sl = pl.ds(kk * 16, 16)
                            s_b[r, sl] = g_b[r, sl] * sp

                @pl.when(jj + 2 < k_chunks)
                def _():
                    start_gather(jj + 2, b)

                start_scatter(jj, b)

        wait_scatter(0)
        wait_scatter(1)
        plsc.subcore_barrier()

        pltpu.sync_copy(acc.at[pl.ds(zbase, rows_per_tile)],
                        out_hbm.at[c].at[pl.ds(zbase, rows_per_tile)])

    return sc_kernel(xt, cols1, rows1, vals1)


@jax.jit
def kernel(x_affine, W_rows, W_cols, W_vals):
    nnz = W_rows.shape[0]
    per_step = NW * CHUNK
    k_chunks = (nnz + per_step - 1) // per_step
    nnz_pad = k_chunks * per_step
    pad = nnz_pad - nnz

    pad_idx = (jnp.arange(pad, dtype=jnp.int32) * 101) % N
    cols1 = jnp.concatenate([W_cols.astype(jnp.int32), pad_idx])
    rows1 = jnp.concatenate([W_rows.astype(jnp.int32), pad_idx])
    vals1 = jnp.concatenate([W_vals, jnp.zeros((pad,), jnp.float32)])

    xt = x_affine.T
    partials = _sc_spmm(xt, cols1, rows1, vals1, k_chunks)
    return _merge_out(partials)


# speedup vs baseline: 21.6489x; 1.1573x over previous
import functools

import jax
import jax.numpy as jnp
from jax import lax
from jax.experimental import pallas as pl
from jax.experimental.pallas import tpu as pltpu
from jax.experimental.pallas import tpu_sc as plsc

N = 16384
B = 64
NC = 2
NS = 16
NW = NC * NS
CHUNK = 128


def _xpose_body(a_ref, b_ref, o_ref):
    o_ref[...] = jnp.concatenate([a_ref[...].T, b_ref[...].T], axis=1)


def _transpose_in(x):
    blk = 1024
    return pl.pallas_call(
        _xpose_body,
        grid=(N // 2 // blk,),
        in_specs=[pl.BlockSpec((B, blk), lambda i: (0, i)),
                  pl.BlockSpec((B, blk), lambda i: (0, i + N // 2 // blk))],
        out_specs=pl.BlockSpec((blk, 2 * B), lambda i: (i, 0)),
        out_shape=jax.ShapeDtypeStruct((N // 2, 2 * B), jnp.float32),
    )(x, x)


def _merge_body(p_ref, oa_ref, ob_ref):
    q = p_ref[0] + p_ref[1]
    oa_ref[...] = q[:, :B].T
    ob_ref[...] = q[:, B:].T


def _merge_out(p):
    blk = 1024
    return pl.pallas_call(
        _merge_body,
        grid=(N // 2 // blk,),
        in_specs=[pl.BlockSpec((NC, blk, 2 * B), lambda i: (0, i, 0))],
        out_specs=[pl.BlockSpec((B, blk), lambda i: (0, i)),
                   pl.BlockSpec((B, blk), lambda i: (0, i))],
        out_shape=[jax.ShapeDtypeStruct((B, N // 2), jnp.float32),
                   jax.ShapeDtypeStruct((B, N // 2), jnp.float32)],
    )(p)


def _splat(v16, i):
    return jnp.broadcast_to(v16[i], (16,))


def _sc_spmm(xt, cols1, rows1, vals1, k_chunks):
    mesh = plsc.VectorSubcoreMesh(core_axis_name="c", subcore_axis_name="s")
    rows_per_tile = N // NS
    t_nnz = k_chunks * CHUNK

    @functools.partial(
        pl.kernel,
        out_type=jax.ShapeDtypeStruct((NC, N, B), jnp.float32),
        mesh=mesh,
        scratch_types=[
            pltpu.VMEM((t_nnz,), jnp.int32),
            pltpu.VMEM((k_chunks, CHUNK), jnp.int32),
            pltpu.VMEM((t_nnz,), jnp.float32),
            pltpu.VMEM((2, CHUNK, B), jnp.float32),
            pltpu.VMEM((2, CHUNK, B), jnp.float32),
            pltpu.VMEM_SHARED((N, B), jnp.float32),
            pltpu.SemaphoreType.DMA,
            pltpu.SemaphoreType.DMA,
            pltpu.SemaphoreType.DMA,
            pltpu.SemaphoreType.DMA,
            pltpu.SemaphoreType.DMA,
            pltpu.SemaphoreType.DMA,
        ],
        compiler_params=pltpu.CompilerParams(use_tc_tiling_on_sc=False),
    )
    def sc_kernel(xt_hbm, cols_hbm, rows_hbm, vals_hbm, out_hbm,
                  cols_v, rows_v, vals_v, gath_v, scat_v, acc,
                  sem_st, zsem, gsem0, gsem1, ssem0, ssem1):
        gsem = (gsem0, gsem1)
        ssem = (ssem0, ssem1)
        c = lax.axis_index("c")
        s = lax.axis_index("s")
        wid = c * NS + s
        base = wid * t_nnz

        pltpu.async_copy(cols_hbm.at[pl.ds(base, t_nnz)], cols_v, sem_st)
        pltpu.async_copy(vals_hbm.at[pl.ds(base, t_nnz)], vals_v, sem_st)

        @pl.loop(0, k_chunks)
        def _(j):
            pltpu.async_copy(rows_hbm.at[pl.ds(base + j * CHUNK, CHUNK)],
                             rows_v.at[j], sem_st)

        zb = scat_v.at[0]

        @pl.loop(0, CHUNK)
        def _(i):
            for kk in range(B // 16):
                zb[i, pl.ds(kk * 16, 16)] = jnp.zeros((16,), jnp.float32)

        zbase = s * rows_per_tile
        n_z = rows_per_tile // CHUNK
        for q in range(n_z):
            pltpu.async_copy(zb, acc.at[pl.ds(zbase + q * CHUNK, CHUNK)], zsem)

        pltpu.make_async_copy(cols_hbm.at[pl.ds(base, t_nnz)], cols_v,
                              sem_st).wait()
        pltpu.make_async_copy(cols_hbm.at[pl.ds(base, t_nnz)], vals_v,
                              sem_st).wait()

        @pl.loop(0, k_chunks)
        def _(j):
            pltpu.make_async_copy(rows_hbm.at[pl.ds(base, CHUNK)],
                                  rows_v.at[0], sem_st).wait()

        def start_gather(j, b):
            pltpu.async_copy(xt_hbm.at[cols_v.at[pl.ds(j * CHUNK, CHUNK)]],
                             gath_v.at[b], gsem[b])

        def wait_gather(b):
            pltpu.make_async_copy(xt_hbm.at[cols_v.at[pl.ds(0, CHUNK)]],
                                  gath_v.at[b], gsem[b]).wait()

        def start_scatter(j, b):
            pltpu.async_copy(scat_v.at[b], acc.at[rows_v.at[j]], ssem[b],
                             add=True)

        def wait_scatter(b):
            pltpu.make_async_copy(xt_hbm.at[cols_v.at[pl.ds(0, CHUNK)]],
                                  scat_v.at[b], ssem[b]).wait()

        start_gather(0, 0)
        start_gather(1, 1)

        for q in range(n_z):
            pltpu.make_async_copy(zb, acc.at[pl.ds(zbase, CHUNK)], zsem).wait()
        plsc.subcore_barrier()

        @pl.loop(0, k_chunks, step=2)
        def _(j):
            for b in range(2):
                jj = j + b
                wait_gather(b)

                @pl.when(jj >= 2)
                def _():
                    wait_scatter(b)

                g_b = gath_v.at[b]
                s_b = scat_v.at[b]

                @pl.loop(0, CHUNK // 16)
                def _(g):
                    v16 = vals_v[pl.ds(jj * CHUNK + g * 16, 16)]
                    for i in range(16):
                        sp = _splat(v16, i)
                        r = g * 16 + i
                        for kk in range(B // 16):
                            sl = pl.ds(kk * 16, 16)
                            s_b[r, sl] = g_b[r, sl] * sp

                @pl.when(jj + 2 < k_chunks)
                def _():
                    start_gather(jj + 2, b)

                start_scatter(jj, b)

        wait_scatter(0)
        wait_scatter(1)
        plsc.subcore_barrier()

        pltpu.sync_copy(acc.at[pl.ds(zbase, rows_per_tile)],
                        out_hbm.at[c].at[pl.ds(zbase, rows_per_tile)])

    return sc_kernel(xt, cols1, rows1, vals1)


@jax.jit
def kernel(x_affine, W_rows, W_cols, W_vals):
    nnz = W_rows.shape[0]
    per_step = NW * CHUNK
    k_chunks = (nnz + per_step - 1) // per_step
    nnz_pad = k_chunks * per_step
    pad = nnz_pad - nnz

    pad_idx = (jnp.arange(pad, dtype=jnp.int32) * 101) % N

    def remap(i):
        return ((i << 1) & (N - 1)) | (i >> 13)

    cols1 = remap(jnp.concatenate([W_cols.astype(jnp.int32), pad_idx]))
    rows1 = remap(jnp.concatenate([W_rows.astype(jnp.int32), pad_idx]))
    vals1 = jnp.concatenate([W_vals, jnp.zeros((pad,), jnp.float32)])

    xt = _transpose_in(x_affine).reshape(N, B)
    partials = _sc_spmm(xt, cols1, rows1, vals1, k_chunks)
    oa, ob = _merge_out(partials.reshape(NC, N // 2, 2 * B))
    return jnp.concatenate([oa, ob], axis=1)


# speedup vs baseline: 22.4420x; 1.0366x over previous
import functools

import jax
import jax.numpy as jnp
from jax import lax
from jax.experimental import pallas as pl
from jax.experimental.pallas import tpu as pltpu
from jax.experimental.pallas import tpu_sc as plsc

N = 16384
B = 64
NC = 2
NS = 16
NW = NC * NS
CHUNK = 128
DEPTH = 3
ROWD = 2 * DEPTH


def _xpose_body(a_ref, b_ref, o_ref):
    o_ref[...] = jnp.concatenate([a_ref[...].T, b_ref[...].T], axis=1)


def _transpose_in(x):
    blk = 1024
    return pl.pallas_call(
        _xpose_body,
        grid=(N // 2 // blk,),
        in_specs=[pl.BlockSpec((B, blk), lambda i: (0, i)),
                  pl.BlockSpec((B, blk), lambda i: (0, i + N // 2 // blk))],
        out_specs=pl.BlockSpec((blk, 2 * B), lambda i: (i, 0)),
        out_shape=jax.ShapeDtypeStruct((N // 2, 2 * B), jnp.float32),
    )(x, x)


_MERGE_NB = 8


def _merge_body(p_ref, o_ref):
    i = pl.program_id(0)
    q = p_ref[0] + p_ref[1]

    @pl.when(i < _MERGE_NB)
    def _():
        o_ref[...] = q[:, :B].T

    @pl.when(i >= _MERGE_NB)
    def _():
        o_ref[...] = q[:, B:].T


def _merge_out(p):
    blk = N // 2 // _MERGE_NB
    return pl.pallas_call(
        _merge_body,
        grid=(2 * _MERGE_NB,),
        in_specs=[pl.BlockSpec((NC, blk, 2 * B),
                               lambda i: (0, lax.rem(i, _MERGE_NB), 0))],
        out_specs=pl.BlockSpec((B, blk), lambda i: (0, i)),
        out_shape=jax.ShapeDtypeStruct((B, N), jnp.float32),
    )(p)


def _splat(v16, i):
    return jnp.broadcast_to(v16[i], (16,))


def _sc_spmm(xt, cols1, rows1, vals1, k_chunks):
    mesh = plsc.VectorSubcoreMesh(core_axis_name="c", subcore_axis_name="s")
    rows_per_tile = N // NS
    t_nnz = k_chunks * CHUNK

    @functools.partial(
        pl.kernel,
        out_type=jax.ShapeDtypeStruct((NC, N, B), jnp.float32),
        mesh=mesh,
        scratch_types=[
            pltpu.VMEM((t_nnz,), jnp.int32),
            pltpu.VMEM((ROWD, CHUNK), jnp.int32),
            pltpu.VMEM((DEPTH, CHUNK), jnp.float32),
            pltpu.VMEM((DEPTH, CHUNK, B), jnp.float32),
            pltpu.VMEM((DEPTH, CHUNK, B), jnp.float32),
            pltpu.VMEM_SHARED((N, B), jnp.float32),
            pltpu.SemaphoreType.DMA,
            pltpu.SemaphoreType.DMA,
            pltpu.SemaphoreType.DMA,
            pltpu.SemaphoreType.DMA,
            pltpu.SemaphoreType.DMA,
            pltpu.SemaphoreType.DMA,
            pltpu.SemaphoreType.DMA,
            pltpu.SemaphoreType.DMA,
        ],
        compiler_params=pltpu.CompilerParams(use_tc_tiling_on_sc=False),
    )
    def sc_kernel(xt_hbm, cols_hbm, rows_hbm, vals_hbm, out_hbm,
                  cols_v, rows_v, vals_v, gath_v, scat_v, acc,
                  sem_st, zsem, gsem0, gsem1, gsem2, ssem0, ssem1, ssem2):
        gsem = (gsem0, gsem1, gsem2)
        ssem = (ssem0, ssem1, ssem2)
        c = lax.axis_index("c")
        s = lax.axis_index("s")
        wid = c * NS + s
        base = wid * t_nnz

        pltpu.async_copy(cols_hbm.at[pl.ds(base, t_nnz)], cols_v, sem_st)

        zb = scat_v.at[0]

        @pl.loop(0, CHUNK)
        def _(i):
            for kk in range(B // 16):
                zb[i, pl.ds(kk * 16, 16)] = jnp.zeros((16,), jnp.float32)

        zbase = s * rows_per_tile
        n_z = rows_per_tile // CHUNK
        for q in range(n_z):
            pltpu.async_copy(zb, acc.at[pl.ds(zbase + q * CHUNK, CHUNK)], zsem)

        pltpu.make_async_copy(cols_hbm.at[pl.ds(base, t_nnz)], cols_v,
                              sem_st).wait()

        def start_gather(j, b, br):
            pltpu.async_copy(rows_hbm.at[pl.ds(base + j * CHUNK, CHUNK)],
                             rows_v.at[br], gsem[b])
            pltpu.async_copy(vals_hbm.at[pl.ds(base + j * CHUNK, CHUNK)],
                             vals_v.at[b], gsem[b])
            pltpu.async_copy(xt_hbm.at[cols_v.at[pl.ds(j * CHUNK, CHUNK)]],
                             gath_v.at[b], gsem[b])

        def wait_gather(b, br):
            pltpu.make_async_copy(rows_hbm.at[pl.ds(base, CHUNK)],
                                  rows_v.at[br], gsem[b]).wait()
            pltpu.make_async_copy(rows_hbm.at[pl.ds(base, CHUNK)],
                                  vals_v.at[b], gsem[b]).wait()
            pltpu.make_async_copy(xt_hbm.at[cols_v.at[pl.ds(0, CHUNK)]],
                                  gath_v.at[b], gsem[b]).wait()

        def start_scatter(b, br):
            pltpu.async_copy(scat_v.at[b], acc.at[rows_v.at[br]], ssem[b],
                             add=True)

        def wait_scatter(b):
            pltpu.make_async_copy(xt_hbm.at[cols_v.at[pl.ds(0, CHUNK)]],
                                  scat_v.at[b], ssem[b]).wait()

        for b in range(DEPTH):
            start_gather(b, b, b)

        for q in range(n_z):
            pltpu.make_async_copy(zb, acc.at[pl.ds(zbase, CHUNK)], zsem).wait()
        plsc.subcore_barrier()

        @pl.loop(0, k_chunks, step=ROWD)
        def _(j):
            for br in range(ROWD):
                b = br % DEPTH
                jj = j + br
                wait_gather(b, br)

                @pl.when(jj >= DEPTH)
                def _():
                    wait_scatter(b)

                g_b = gath_v.at[b]
                s_b = scat_v.at[b]

                @pl.loop(0, CHUNK // 16)
                def _(g):
                    v16 = vals_v[b, pl.ds(g * 16, 16)]
                    for i in range(16):
                        sp = _splat(v16, i)
                        r = g * 16 + i
                        for kk in range(B // 16):
                            sl = pl.ds(kk * 16, 16)
                            s_b[r, sl] = g_b[r, sl] * sp

                @pl.when(jj + DEPTH < k_chunks)
                def _():
                    start_gather(jj + DEPTH, b, (br + DEPTH) % ROWD)

                start_scatter(b, br)

        for b in range(DEPTH):
            wait_scatter(b)
        plsc.subcore_barrier()

        pltpu.sync_copy(acc.at[pl.ds(zbase, rows_per_tile)],
                        out_hbm.at[c].at[pl.ds(zbase, rows_per_tile)])

    return sc_kernel(xt, cols1, rows1, vals1)


@jax.jit
def kernel(x_affine, W_rows, W_cols, W_vals):
    nnz = W_rows.shape[0]
    per_step = NW * CHUNK
    k_chunks = (nnz + per_step - 1) // per_step
    nnz_pad = k_chunks * per_step
    pad = nnz_pad - nnz

    pad_idx = (jnp.arange(pad, dtype=jnp.int32) * 101) % N

    def remap(i):
        return ((i << 1) & (N - 1)) | (i >> 13)

    cols1 = remap(jnp.concatenate([W_cols.astype(jnp.int32), pad_idx]))
    rows1 = remap(jnp.concatenate([W_rows.astype(jnp.int32), pad_idx]))
    vals1 = jnp.concatenate([W_vals, jnp.zeros((pad,), jnp.float32)])

    xt = _transpose_in(x_affine).reshape(N, B)
    partials = _sc_spmm(xt, cols1, rows1, vals1, k_chunks)
    return _merge_out(partials.reshape(NC, N // 2, 2 * B))
